# baseline (speedup 1.0000x reference)
import functools

import jax
import jax.numpy as jnp
from jax import lax
from jax.experimental import pallas as pl
from jax.experimental.pallas import tpu as pltpu
from jax.experimental.pallas import tpu_sc as plsc

N = 10000
E = 60000
B = 64
IN = 1280
ED = 16
HID = 256

EP = 61440
GW = 48
SW = 64
BN = 400
BE1 = 512
BE4 = 1024

_f32 = jnp.float32


def _make_gather(d, mesh, dtype=_f32):
    chunk = EP // 32
    nwin = chunk // GW

    @functools.partial(
        pl.kernel,
        out_type=jax.ShapeDtypeStruct((EP, d), dtype),
        mesh=mesh,
        scratch_types=[
            pltpu.VMEM((nwin, GW), jnp.int32),
            pltpu.VMEM((GW, d), dtype),
            pltpu.SemaphoreType.DMA,
        ],
    )
    def gk(tbl_hbm, src2d_hbm, out_hbm, idx_v, rows_v, sem):
        c = lax.axis_index("c")
        s = lax.axis_index("s")
        wid = s * 2 + c
        ebase = wid * chunk
        pltpu.sync_copy(src2d_hbm.at[pl.ds(wid * nwin, nwin)], idx_v)

        def body(g, carry):
            pltpu.async_copy(tbl_hbm.at[idx_v.at[g]], rows_v, sem).wait()
            pltpu.sync_copy(rows_v, out_hbm.at[pl.ds(ebase + g * GW, GW)])
            return carry

        lax.fori_loop(0, nwin, body, 0)

    return gk


NROW = 10112
ZR = NROW // 16
HC = HID // 2


def _make_scatter(mesh):
    nwin = EP // 16 // SW

    @functools.partial(
        pl.kernel,
        out_type=jax.ShapeDtypeStruct((NROW, HID), _f32),
        mesh=mesh,
        scratch_types=[
            pltpu.VMEM((nwin, SW), jnp.int32),
            pltpu.VMEM((SW, HC), _f32),
            pltpu.VMEM_SHARED((NROW, HC), _f32),
        ],
    )
    def _scatter_kernel(msg_hbm, dst3_hbm, zero_hbm, agg_hbm,
                        idxbuf, updbuf, acc):
        c = lax.axis_index("c")
        s = lax.axis_index("s")
        ebase = s * (EP // 16)
        col = c * HC
        pltpu.sync_copy(dst3_hbm.at[s], idxbuf)
        pltpu.sync_copy(zero_hbm, acc.at[pl.ds(s * ZR, ZR)])
        plsc.subcore_barrier()

        def body(g, carry):
            pltpu.sync_copy(
                msg_hbm.at[pl.ds(ebase + g * SW, SW), pl.ds(col, HC)],
                updbuf)
            pltpu.sync_copy(updbuf, acc.at[idxbuf.at[g]], add=True)
            return carry

        lax.fori_loop(0, nwin, body, 0)
        plsc.subcore_barrier()
        pltpu.sync_copy(
            acc.at[pl.ds(s * ZR, ZR)],
            agg_hbm.at[pl.ds(s * ZR, ZR), pl.ds(col, HC)])

    return _scatter_kernel


@functools.lru_cache(maxsize=1)
def _sc_kernels():
    mesh = plsc.VectorSubcoreMesh(core_axis_name="c", subcore_axis_name="s")
    return (_make_gather(IN // 2, mesh, jnp.int32), _make_gather(HID, mesh),
            _make_scatter(mesh))


def _sel_block(batch_ref, pos_ref, center_ref):
    b = batch_ref[0, 0, :][None, :]
    p = pos_ref[0, 0, :][None, :]
    ci = center_ref[:, 0:1]
    ii = lax.broadcasted_iota(jnp.int32, (B, BN), 0)
    return ((b == ii) & (p == ci)).astype(_f32)


def _edge0_body(xg_ref, ea_ref, wea_ref, web_ref, bea_ref, beb_ref,
                w1a_ref, w1b_ref, m_ref):
    v = xg_ref[...]
    a = lax.bitcast_convert_type(v << 16, _f32)
    b = lax.bitcast_convert_type(v & jnp.int32(-65536), _f32)
    ea = ea_ref[...]
    ta = jnp.maximum(
        a + jnp.dot(ea, wea_ref[...], preferred_element_type=_f32)
        + bea_ref[...], 0.0)
    tb = jnp.maximum(
        b + jnp.dot(ea, web_ref[...], preferred_element_type=_f32)
        + beb_ref[...], 0.0)
    m_ref[...] = (jnp.dot(ta, w1a_ref[...], preferred_element_type=_f32)
                  + jnp.dot(tb, w1b_ref[...], preferred_element_type=_f32))


def _edge0(xg, ea, wea, web, bea, beb, w1a, w1b):
    half = pl.BlockSpec((ED, IN // 2), lambda i: (0, 0))
    vec = pl.BlockSpec((1, IN // 2), lambda i: (0, 0))
    mat = pl.BlockSpec((IN // 2, HID), lambda i: (0, 0))
    return pl.pallas_call(
        _edge0_body,
        grid=(EP // BE1,),
        in_specs=[
            pl.BlockSpec((BE1, IN // 2), lambda i: (i, 0)),
            pl.BlockSpec((BE1, ED), lambda i: (i, 0)),
            half, half, vec, vec, mat, mat,
        ],
        out_specs=pl.BlockSpec((BE1, HID), lambda i: (i, 0)),
        out_shape=jax.ShapeDtypeStruct((EP, HID), _f32),
    )(xg, ea, wea, web, bea, beb, w1a, w1b)


def _edge_body(hg_ref, ea_ref, we_ref, be_ref, m_ref):
    emb = jnp.dot(ea_ref[...], we_ref[...], preferred_element_type=_f32)
    m_ref[...] = jnp.maximum(hg_ref[...] + emb + be_ref[...], 0.0)


def _edge(hg, ea, we, be_):
    return pl.pallas_call(
        _edge_body,
        grid=(EP // BE4,),
        in_specs=[
            pl.BlockSpec((BE4, HID), lambda i: (i, 0)),
            pl.BlockSpec((BE4, ED), lambda i: (i, 0)),
            pl.BlockSpec((ED, HID), lambda i: (0, 0)),
            pl.BlockSpec((1, HID), lambda i: (0, 0)),
        ],
        out_specs=pl.BlockSpec((BE4, HID), lambda i: (i, 0)),
        out_shape=jax.ShapeDtypeStruct((EP, HID), _f32),
    )(hg, ea, we, be_)


def _xw1_esm_body(x_ref, w1_ref, batch_ref, pos_ref, center_ref,
                  xw_ref, esm_ref):
    i = pl.program_id(0)
    xb = x_ref[...]
    xw_ref[...] = jnp.dot(xb, w1_ref[...], preferred_element_type=_f32)
    sel = _sel_block(batch_ref, pos_ref, center_ref)

    @pl.when(i == 0)
    def _():
        esm_ref[...] = jnp.zeros_like(esm_ref)

    esm_ref[...] += jnp.dot(sel, xb, preferred_element_type=_f32)


def _xw1_esm(x, w1, batch3, pos3, center2d):
    return pl.pallas_call(
        _xw1_esm_body,
        grid=(N // BN,),
        in_specs=[
            pl.BlockSpec((BN, IN), lambda i: (i, 0)),
            pl.BlockSpec((IN, HID), lambda i: (0, 0)),
            pl.BlockSpec((1, 1, BN), lambda i: (i, 0, 0)),
            pl.BlockSpec((1, 1, BN), lambda i: (i, 0, 0)),
            pl.BlockSpec((B, 128), lambda i: (0, 0)),
        ],
        out_specs=[
            pl.BlockSpec((BN, HID), lambda i: (i, 0)),
            pl.BlockSpec((B, IN), lambda i: (0, 0)),
        ],
        out_shape=[
            jax.ShapeDtypeStruct((N, HID), _f32),
            jax.ShapeDtypeStruct((B, IN), _f32),
        ],
    )(x, w1, batch3, pos3, center2d)


def _mlp_ln_tail(u, w2_ref, b2_ref, g_ref, bb_ref, out_ref):
    v = jnp.dot(u, w2_ref[...], preferred_element_type=_f32) + b2_ref[...]
    r = jnp.maximum(v, 0.0)
    mu = jnp.mean(r, axis=1, keepdims=True)
    var = jnp.mean((r - mu) * (r - mu), axis=1, keepdims=True)
    out_ref[...] = (r - mu) * lax.rsqrt(var + 1e-5) * g_ref[...] + bb_ref[...]


def _node0_body(xw_ref, agg_ref, b1_ref, w2_ref, b2_ref, g_ref, bb_ref,
                out_ref):
    u = jnp.maximum(xw_ref[...] + agg_ref[...] + b1_ref[...], 0.0)
    _mlp_ln_tail(u, w2_ref, b2_ref, g_ref, bb_ref, out_ref)


def _node0(xw, agg, b1, w2, b2, g, bb):
    vec = pl.BlockSpec((1, HID), lambda i: (0, 0))
    return pl.pallas_call(
        _node0_body,
        grid=(N // BN,),
        in_specs=[
            pl.BlockSpec((BN, HID), lambda i: (i, 0)),
            pl.BlockSpec((BN, HID), lambda i: (i, 0)),
            vec, pl.BlockSpec((HID, HID), lambda i: (0, 0)), vec, vec, vec,
        ],
        out_specs=pl.BlockSpec((BN, HID), lambda i: (i, 0)),
        out_shape=jax.ShapeDtypeStruct((N, HID), _f32),
    )(xw, agg, b1, w2, b2, g, bb)


def _node_body(h_ref, agg_ref, w1_ref, b1_ref, w2_ref, b2_ref, g_ref, bb_ref,
               out_ref):
    z = h_ref[...] + agg_ref[...]
    u = jnp.maximum(
        jnp.dot(z, w1_ref[...], preferred_element_type=_f32) + b1_ref[...],
        0.0)
    _mlp_ln_tail(u, w2_ref, b2_ref, g_ref, bb_ref, out_ref)


def _node(h, agg, w1, b1, w2, b2, g, bb):
    vec = pl.BlockSpec((1, HID), lambda i: (0, 0))
    mat = pl.BlockSpec((HID, HID), lambda i: (0, 0))
    return pl.pallas_call(
        _node_body,
        grid=(N // BN,),
        in_specs=[
            pl.BlockSpec((BN, HID), lambda i: (i, 0)),
            pl.BlockSpec((BN, HID), lambda i: (i, 0)),
            mat, vec, mat, vec, vec, vec,
        ],
        out_specs=pl.BlockSpec((BN, HID), lambda i: (i, 0)),
        out_shape=jax.ShapeDtypeStruct((N, HID), _f32),
    )(h, agg, w1, b1, w2, b2, g, bb)


def _head_body(h_ref, batch_ref, pos_ref, center_ref, esm_ref,
               wg_ref, we_ref, bf1_ref, wf2_ref, bf2_ref,
               out_ref, acc_ref):
    i = pl.program_id(0)
    sel = _sel_block(batch_ref, pos_ref, center_ref)

    @pl.when(i == 0)
    def _():
        acc_ref[...] = jnp.zeros_like(acc_ref)

    acc_ref[...] += jnp.dot(sel, h_ref[...], preferred_element_type=_f32)

    @pl.when(i == N // BN - 1)
    def _():
        g = (jnp.dot(acc_ref[...], wg_ref[...], preferred_element_type=_f32)
             + jnp.dot(esm_ref[...], we_ref[...], preferred_element_type=_f32)
             + bf1_ref[...])
        r = jnp.maximum(g, 0.0)
        out_ref[...] = (jnp.dot(r, wf2_ref[...], preferred_element_type=_f32)
                        + bf2_ref[...])


def _head(h3, batch3, pos3, center2d, esm, wf1g, wf1e, bf1, wf2p, bf2p):
    return pl.pallas_call(
        _head_body,
        grid=(N // BN,),
        in_specs=[
            pl.BlockSpec((BN, HID), lambda i: (i, 0)),
            pl.BlockSpec((1, 1, BN), lambda i: (i, 0, 0)),
            pl.BlockSpec((1, 1, BN), lambda i: (i, 0, 0)),
            pl.BlockSpec((B, 128), lambda i: (0, 0)),
            pl.BlockSpec((B, IN), lambda i: (0, 0)),
            pl.BlockSpec((HID, HID), lambda i: (0, 0)),
            pl.BlockSpec((IN, HID), lambda i: (0, 0)),
            pl.BlockSpec((1, HID), lambda i: (0, 0)),
            pl.BlockSpec((HID, 128), lambda i: (0, 0)),
            pl.BlockSpec((1, 128), lambda i: (0, 0)),
        ],
        out_specs=pl.BlockSpec((B, 128), lambda i: (0, 0)),
        out_shape=jax.ShapeDtypeStruct((B, 128), _f32),
        scratch_shapes=[pltpu.VMEM((B, HID), _f32)],
    )(h3, batch3, pos3, center2d, esm, wf1g, wf1e, bf1, wf2p, bf2p)


def kernel(x, edge_index, edge_attr, batch, pos_idx, center_idx, params):
    pad = EP - E
    src = edge_index[0]
    dst = edge_index[1]
    src2d = jnp.concatenate(
        [src, jnp.zeros((pad,), jnp.int32)]).reshape(EP // GW, GW)
    dst2 = jnp.concatenate(
        [dst, N + (jnp.arange(pad, dtype=jnp.int32) & 7)]
    ).reshape(16, EP // 16 // SW, SW)
    zrows = jnp.zeros((ZR, HC), _f32)
    ea_pad = jnp.concatenate([edge_attr, jnp.zeros((pad, ED), _f32)])
    batch3 = batch.reshape(N // BN, 1, BN)
    pos3 = pos_idx.reshape(N // BN, 1, BN)
    center2d = jnp.broadcast_to(center_idx[:, None], (B, 128))

    L = params['layers']
    r1 = lambda a: a.reshape(1, -1)
    _gather_x, _gather_h, _scatter_add = _sc_kernels()

    xi = lax.bitcast_convert_type(x, jnp.int32) + jnp.int32(0x8000)
    xpk = (lax.shift_right_logical(xi[:, 0::2], 16)
           | (xi[:, 1::2] & jnp.int32(-65536)))
    xg = _gather_x(xpk, src2d)
    We0, be0, W10 = L[0]['We'], L[0]['be'], L[0]['W1']
    m0 = _edge0(xg, ea_pad, We0[:, 0::2], We0[:, 1::2],
                r1(be0[0::2]), r1(be0[1::2]), W10[0::2], W10[1::2])
    agg = _scatter_add(m0, dst2, zrows)[:N]
    xw, esm = _xw1_esm(x, L[0]['W1'], batch3, pos3, center2d)
    h = _node0(xw, agg, r1(L[0]['b1']), L[0]['W2'], r1(L[0]['b2']),
               r1(L[0]['gamma']), r1(L[0]['beta']))

    for p in L[1:]:
        hg = _gather_h(h, src2d)
        msg = _edge(hg, ea_pad, p['We'], r1(p['be']))
        agg = _scatter_add(msg, dst2, zrows)[:N]
        h = _node(h, agg, p['W1'], r1(p['b1']), p['W2'], r1(p['b2']),
                  r1(p['gamma']), r1(p['beta']))

    wf2p = jnp.pad(params['Wf2'], ((0, 0), (0, 126)))
    bf2p = jnp.pad(params['bf2'], (0, 126)).reshape(1, 128)
    out = _head(h, batch3, pos3, center2d, esm,
                params['Wf1'][:HID], params['Wf1'][HID:],
                r1(params['bf1']), wf2p, bf2p)
    return out[:, :2]


# speedup vs baseline: 2.2166x; 2.2166x over previous
import functools

import jax
import jax.numpy as jnp
from jax import lax
from jax.experimental import pallas as pl
from jax.experimental.pallas import tpu as pltpu
from jax.experimental.pallas import tpu_sc as plsc

N = 10000
E = 60000
B = 64
IN = 1280
ED = 16
HID = 256

EP = 61440
GW = 48
SW = 64
BN = 400
BE1 = 512
BE4 = 1024

_f32 = jnp.float32


def _make_gather(d, mesh, dtype=_f32):
    chunk = EP // 32
    nwin = chunk // GW

    @functools.partial(
        pl.kernel,
        out_type=jax.ShapeDtypeStruct((EP, d), dtype),
        mesh=mesh,
        scratch_types=[
            pltpu.VMEM((nwin, GW), jnp.int32),
            pltpu.VMEM((GW, d), dtype),
            pltpu.SemaphoreType.DMA,
        ],
    )
    def gk(tbl_hbm, src2d_hbm, out_hbm, idx_v, rows_v, sem):
        c = lax.axis_index("c")
        s = lax.axis_index("s")
        wid = s * 2 + c
        ebase = wid * chunk
        pltpu.sync_copy(src2d_hbm.at[pl.ds(wid * nwin, nwin)], idx_v)

        def body(g, carry):
            pltpu.async_copy(tbl_hbm.at[idx_v.at[g]], rows_v, sem).wait()
            pltpu.sync_copy(rows_v, out_hbm.at[pl.ds(ebase + g * GW, GW)])
            return carry

        lax.fori_loop(0, nwin, body, 0)

    return gk


NROW = 10112
ZR = NROW // 16
HC = HID // 2


def _make_scatter(mesh):
    nwin = EP // 16 // SW

    @functools.partial(
        pl.kernel,
        out_type=jax.ShapeDtypeStruct((NROW, HID), _f32),
        mesh=mesh,
        scratch_types=[
            pltpu.VMEM((nwin, SW), jnp.int32),
            pltpu.VMEM((SW, HC), _f32),
            pltpu.VMEM_SHARED((NROW, HC), _f32),
        ],
    )
    def _scatter_kernel(msg_hbm, dst3_hbm, zero_hbm, agg_hbm,
                        idxbuf, updbuf, acc):
        c = lax.axis_index("c")
        s = lax.axis_index("s")
        ebase = s * (EP // 16)
        col = c * HC
        pltpu.sync_copy(dst3_hbm.at[s], idxbuf)
        pltpu.sync_copy(zero_hbm, acc.at[pl.ds(s * ZR, ZR)])
        plsc.subcore_barrier()

        def body(g, carry):
            pltpu.sync_copy(
                msg_hbm.at[pl.ds(ebase + g * SW, SW), pl.ds(col, HC)],
                updbuf)
            pltpu.sync_copy(updbuf, acc.at[idxbuf.at[g]], add=True)
            return carry

        lax.fori_loop(0, nwin, body, 0)
        plsc.subcore_barrier()
        pltpu.sync_copy(
            acc.at[pl.ds(s * ZR, ZR)],
            agg_hbm.at[pl.ds(s * ZR, ZR), pl.ds(col, HC)])

    return _scatter_kernel


@functools.lru_cache(maxsize=1)
def _sc_kernels():
    mesh = plsc.VectorSubcoreMesh(core_axis_name="c", subcore_axis_name="s")
    return (_make_gather(IN // 2, mesh, jnp.int32), _make_gather(HID, mesh),
            _make_scatter(mesh))


def _sel_block(batch_ref, pos_ref, center_ref):
    b = batch_ref[0, 0, :][None, :]
    p = pos_ref[0, 0, :][None, :]
    ci = center_ref[:, 0:1]
    ii = lax.broadcasted_iota(jnp.int32, (B, BN), 0)
    return ((b == ii) & (p == ci)).astype(_f32)


def _edge0_body(xg_ref, ea_ref, wea_ref, web_ref, bea_ref, beb_ref,
                w1a_ref, w1b_ref, m_ref):
    v = xg_ref[...]
    a = lax.bitcast_convert_type(v << 16, _f32)
    b = lax.bitcast_convert_type(v & jnp.int32(-65536), _f32)
    ea = ea_ref[...]
    ta = jnp.maximum(
        a + jnp.dot(ea, wea_ref[...], preferred_element_type=_f32)
        + bea_ref[...], 0.0)
    tb = jnp.maximum(
        b + jnp.dot(ea, web_ref[...], preferred_element_type=_f32)
        + beb_ref[...], 0.0)
    m_ref[...] = (jnp.dot(ta, w1a_ref[...], preferred_element_type=_f32)
                  + jnp.dot(tb, w1b_ref[...], preferred_element_type=_f32))


def _edge0(xg, ea, wea, web, bea, beb, w1a, w1b):
    half = pl.BlockSpec((ED, IN // 2), lambda i: (0, 0))
    vec = pl.BlockSpec((1, IN // 2), lambda i: (0, 0))
    mat = pl.BlockSpec((IN // 2, HID), lambda i: (0, 0))
    return pl.pallas_call(
        _edge0_body,
        grid=(EP // BE1,),
        in_specs=[
            pl.BlockSpec((BE1, IN // 2), lambda i: (i, 0)),
            pl.BlockSpec((BE1, ED), lambda i: (i, 0)),
            half, half, vec, vec, mat, mat,
        ],
        out_specs=pl.BlockSpec((BE1, HID), lambda i: (i, 0)),
        out_shape=jax.ShapeDtypeStruct((EP, HID), _f32),
    )(xg, ea, wea, web, bea, beb, w1a, w1b)


def _edge_body(hg_ref, ea_ref, we_ref, be_ref, m_ref):
    emb = jnp.dot(ea_ref[...], we_ref[...], preferred_element_type=_f32)
    m_ref[...] = jnp.maximum(hg_ref[...] + emb + be_ref[...], 0.0)


def _edge(hg, ea, we, be_):
    return pl.pallas_call(
        _edge_body,
        grid=(EP // BE4,),
        in_specs=[
            pl.BlockSpec((BE4, HID), lambda i: (i, 0)),
            pl.BlockSpec((BE4, ED), lambda i: (i, 0)),
            pl.BlockSpec((ED, HID), lambda i: (0, 0)),
            pl.BlockSpec((1, HID), lambda i: (0, 0)),
        ],
        out_specs=pl.BlockSpec((BE4, HID), lambda i: (i, 0)),
        out_shape=jax.ShapeDtypeStruct((EP, HID), _f32),
    )(hg, ea, we, be_)


def _xw1_esm_body(x_ref, w1_ref, batch_ref, pos_ref, center_ref,
                  xw_ref, esm_ref):
    i = pl.program_id(0)
    xb = x_ref[...]
    xw_ref[...] = jnp.dot(xb, w1_ref[...], preferred_element_type=_f32)
    sel = _sel_block(batch_ref, pos_ref, center_ref)

    @pl.when(i == 0)
    def _():
        esm_ref[...] = jnp.zeros_like(esm_ref)

    esm_ref[...] += jnp.dot(sel, xb, preferred_element_type=_f32)


def _xw1_esm(x, w1, batch3, pos3, center2d):
    return pl.pallas_call(
        _xw1_esm_body,
        grid=(N // BN,),
        in_specs=[
            pl.BlockSpec((BN, IN), lambda i: (i, 0)),
            pl.BlockSpec((IN, HID), lambda i: (0, 0)),
            pl.BlockSpec((1, 1, BN), lambda i: (i, 0, 0)),
            pl.BlockSpec((1, 1, BN), lambda i: (i, 0, 0)),
            pl.BlockSpec((B, 128), lambda i: (0, 0)),
        ],
        out_specs=[
            pl.BlockSpec((BN, HID), lambda i: (i, 0)),
            pl.BlockSpec((B, IN), lambda i: (0, 0)),
        ],
        out_shape=[
            jax.ShapeDtypeStruct((N, HID), _f32),
            jax.ShapeDtypeStruct((B, IN), _f32),
        ],
    )(x, w1, batch3, pos3, center2d)


def _mlp_ln_tail(u, w2_ref, b2_ref, g_ref, bb_ref, out_ref):
    v = jnp.dot(u, w2_ref[...], preferred_element_type=_f32) + b2_ref[...]
    r = jnp.maximum(v, 0.0)
    mu = jnp.mean(r, axis=1, keepdims=True)
    var = jnp.mean((r - mu) * (r - mu), axis=1, keepdims=True)
    out_ref[...] = (r - mu) * lax.rsqrt(var + 1e-5) * g_ref[...] + bb_ref[...]


def _node0_body(xw_ref, agg_ref, b1_ref, w2_ref, b2_ref, g_ref, bb_ref,
                out_ref):
    u = jnp.maximum(xw_ref[...] + agg_ref[...] + b1_ref[...], 0.0)
    _mlp_ln_tail(u, w2_ref, b2_ref, g_ref, bb_ref, out_ref)


def _node0(xw, agg, b1, w2, b2, g, bb):
    vec = pl.BlockSpec((1, HID), lambda i: (0, 0))
    return pl.pallas_call(
        _node0_body,
        grid=(N // BN,),
        in_specs=[
            pl.BlockSpec((BN, HID), lambda i: (i, 0)),
            pl.BlockSpec((BN, HID), lambda i: (i, 0)),
            vec, pl.BlockSpec((HID, HID), lambda i: (0, 0)), vec, vec, vec,
        ],
        out_specs=pl.BlockSpec((BN, HID), lambda i: (i, 0)),
        out_shape=jax.ShapeDtypeStruct((N, HID), _f32),
    )(xw, agg, b1, w2, b2, g, bb)


def _node_body(h_ref, agg_ref, w1_ref, b1_ref, w2_ref, b2_ref, g_ref, bb_ref,
               out_ref):
    z = h_ref[...] + agg_ref[...]
    u = jnp.maximum(
        jnp.dot(z, w1_ref[...], preferred_element_type=_f32) + b1_ref[...],
        0.0)
    _mlp_ln_tail(u, w2_ref, b2_ref, g_ref, bb_ref, out_ref)


def _node(h, agg, w1, b1, w2, b2, g, bb):
    vec = pl.BlockSpec((1, HID), lambda i: (0, 0))
    mat = pl.BlockSpec((HID, HID), lambda i: (0, 0))
    return pl.pallas_call(
        _node_body,
        grid=(N // BN,),
        in_specs=[
            pl.BlockSpec((BN, HID), lambda i: (i, 0)),
            pl.BlockSpec((BN, HID), lambda i: (i, 0)),
            mat, vec, mat, vec, vec, vec,
        ],
        out_specs=pl.BlockSpec((BN, HID), lambda i: (i, 0)),
        out_shape=jax.ShapeDtypeStruct((N, HID), _f32),
    )(h, agg, w1, b1, w2, b2, g, bb)


def _head_body(h_ref, batch_ref, pos_ref, center_ref, esm_ref,
               wg_ref, we_ref, bf1_ref, wf2_ref, bf2_ref,
               out_ref, acc_ref):
    i = pl.program_id(0)
    sel = _sel_block(batch_ref, pos_ref, center_ref)

    @pl.when(i == 0)
    def _():
        acc_ref[...] = jnp.zeros_like(acc_ref)

    acc_ref[...] += jnp.dot(sel, h_ref[...], preferred_element_type=_f32)

    @pl.when(i == N // BN - 1)
    def _():
        g = (jnp.dot(acc_ref[...], wg_ref[...], preferred_element_type=_f32)
             + jnp.dot(esm_ref[...], we_ref[...], preferred_element_type=_f32)
             + bf1_ref[...])
        r = jnp.maximum(g, 0.0)
        out_ref[...] = (jnp.dot(r, wf2_ref[...], preferred_element_type=_f32)
                        + bf2_ref[...])


def _head(h3, batch3, pos3, center2d, esm, wf1g, wf1e, bf1, wf2p, bf2p):
    return pl.pallas_call(
        _head_body,
        grid=(N // BN,),
        in_specs=[
            pl.BlockSpec((BN, HID), lambda i: (i, 0)),
            pl.BlockSpec((1, 1, BN), lambda i: (i, 0, 0)),
            pl.BlockSpec((1, 1, BN), lambda i: (i, 0, 0)),
            pl.BlockSpec((B, 128), lambda i: (0, 0)),
            pl.BlockSpec((B, IN), lambda i: (0, 0)),
            pl.BlockSpec((HID, HID), lambda i: (0, 0)),
            pl.BlockSpec((IN, HID), lambda i: (0, 0)),
            pl.BlockSpec((1, HID), lambda i: (0, 0)),
            pl.BlockSpec((HID, 128), lambda i: (0, 0)),
            pl.BlockSpec((1, 128), lambda i: (0, 0)),
        ],
        out_specs=pl.BlockSpec((B, 128), lambda i: (0, 0)),
        out_shape=jax.ShapeDtypeStruct((B, 128), _f32),
        scratch_shapes=[pltpu.VMEM((B, HID), _f32)],
    )(h3, batch3, pos3, center2d, esm, wf1g, wf1e, bf1, wf2p, bf2p)


def kernel(x, edge_index, edge_attr, batch, pos_idx, center_idx, params):
    pad = EP - E
    src = edge_index[0]
    dst = edge_index[1]
    src2d = jnp.concatenate(
        [src, jnp.zeros((pad,), jnp.int32)]).reshape(EP // GW, GW)
    dst2 = jnp.concatenate(
        [dst, N + (jnp.arange(pad, dtype=jnp.int32) & 7)]
    ).reshape(16, EP // 16 // SW, SW)
    zrows = jnp.zeros((ZR, HC), _f32)
    ea_pad = jnp.concatenate([edge_attr, jnp.zeros((pad, ED), _f32)])
    batch3 = batch.reshape(N // BN, 1, BN)
    pos3 = pos_idx.reshape(N // BN, 1, BN)
    center2d = jnp.broadcast_to(center_idx[:, None], (B, 128))

    L = params['layers']
    r1 = lambda a: a.reshape(1, -1)
    _gather_x, _gather_h, _scatter_add = _sc_kernels()

    H2 = IN // 2
    xi = lax.bitcast_convert_type(x, jnp.int32) + jnp.int32(0x8000)
    xpk = (lax.shift_right_logical(xi[:, :H2], 16)
           | (xi[:, H2:] & jnp.int32(-65536)))
    xg = _gather_x(xpk, src2d)
    We0, be0, W10 = L[0]['We'], L[0]['be'], L[0]['W1']
    m0 = _edge0(xg, ea_pad, We0[:, :H2], We0[:, H2:],
                r1(be0[:H2]), r1(be0[H2:]), W10[:H2], W10[H2:])
    agg = _scatter_add(m0, dst2, zrows)[:N]
    xw, esm = _xw1_esm(x, L[0]['W1'], batch3, pos3, center2d)
    h = _node0(xw, agg, r1(L[0]['b1']), L[0]['W2'], r1(L[0]['b2']),
               r1(L[0]['gamma']), r1(L[0]['beta']))

    for p in L[1:]:
        hg = _gather_h(h, src2d)
        msg = _edge(hg, ea_pad, p['We'], r1(p['be']))
        agg = _scatter_add(msg, dst2, zrows)[:N]
        h = _node(h, agg, p['W1'], r1(p['b1']), p['W2'], r1(p['b2']),
                  r1(p['gamma']), r1(p['beta']))

    wf2p = jnp.pad(params['Wf2'], ((0, 0), (0, 126)))
    bf2p = jnp.pad(params['bf2'], (0, 126)).reshape(1, 128)
    out = _head(h, batch3, pos3, center2d, esm,
                params['Wf1'][:HID], params['Wf1'][HID:],
                r1(params['bf1']), wf2p, bf2p)
    return out[:, :2]


# speedup vs baseline: 2.3978x; 1.0818x over previous
import functools

import jax
import jax.numpy as jnp
from jax import lax
from jax.experimental import pallas as pl
from jax.experimental.pallas import tpu as pltpu
from jax.experimental.pallas import tpu_sc as plsc

N = 10000
E = 60000
B = 64
IN = 1280
ED = 16
HID = 256

EP = 61440
GW = 64
GWH = 96
SW = 128
BN = 400
BE1 = 512
BE4 = 1024

_f32 = jnp.float32


def _make_gather(d, mesh, dtype=_f32, gw=GW):
    chunk = EP // 32
    nwin = chunk // gw

    @functools.partial(
        pl.kernel,
        out_type=jax.ShapeDtypeStruct((EP, d), dtype),
        mesh=mesh,
        scratch_types=[
            pltpu.VMEM((nwin, gw), jnp.int32),
            pltpu.VMEM((gw, d), dtype),
            pltpu.SemaphoreType.DMA,
        ],
    )
    def gk(tbl_hbm, src3_hbm, out_hbm, idx_v, rows_v, sem):
        c = lax.axis_index("c")
        s = lax.axis_index("s")
        wid = s * 2 + c
        ebase = wid * chunk
        pltpu.sync_copy(src3_hbm.at[wid], idx_v)

        def body(g, carry):
            pltpu.async_copy(tbl_hbm.at[idx_v.at[g]], rows_v, sem).wait()
            pltpu.sync_copy(rows_v, out_hbm.at[pl.ds(ebase + g * gw, gw)])
            return carry

        lax.fori_loop(0, nwin, body, 0)

    return gk


NROW = 10112
ZR = NROW // 16
HC = HID // 2


def _make_scatter(mesh):
    nwin = EP // 16 // SW

    @functools.partial(
        pl.kernel,
        out_type=jax.ShapeDtypeStruct((NROW, HID), _f32),
        mesh=mesh,
        scratch_types=[
            pltpu.VMEM((nwin, SW), jnp.int32),
            pltpu.VMEM((SW, HC), _f32),
            pltpu.VMEM_SHARED((NROW, HC), _f32),
        ],
    )
    def _scatter_kernel(msg_hbm, dst3_hbm, zero_hbm, agg_hbm,
                        idxbuf, updbuf, acc):
        c = lax.axis_index("c")
        s = lax.axis_index("s")
        ebase = s * (EP // 16)
        col = c * HC
        pltpu.sync_copy(dst3_hbm.at[s], idxbuf)
        pltpu.sync_copy(zero_hbm, acc.at[pl.ds(s * ZR, ZR)])
        plsc.subcore_barrier()

        def body(g, carry):
            pltpu.sync_copy(
                msg_hbm.at[pl.ds(ebase + g * SW, SW), pl.ds(col, HC)],
                updbuf)
            pltpu.sync_copy(updbuf, acc.at[idxbuf.at[g]], add=True)
            return carry

        lax.fori_loop(0, nwin, body, 0)
        plsc.subcore_barrier()
        pltpu.sync_copy(
            acc.at[pl.ds(s * ZR, ZR)],
            agg_hbm.at[pl.ds(s * ZR, ZR), pl.ds(col, HC)])

    return _scatter_kernel


@functools.lru_cache(maxsize=1)
def _sc_kernels():
    mesh = plsc.VectorSubcoreMesh(core_axis_name="c", subcore_axis_name="s")
    return (_make_gather(IN // 2, mesh, jnp.int32, GW),
            _make_gather(HID, mesh, _f32, GWH),
            _make_scatter(mesh))


def _sel_block(batch_ref, pos_ref, center_ref):
    b = batch_ref[0, 0, :][None, :]
    p = pos_ref[0, 0, :][None, :]
    ci = center_ref[:, 0:1]
    ii = lax.broadcasted_iota(jnp.int32, (B, BN), 0)
    return ((b == ii) & (p == ci)).astype(_f32)


def _edge0_body(xg_ref, ea_ref, wea_ref, web_ref, bea_ref, beb_ref,
                w1a_ref, w1b_ref, m_ref):
    v = xg_ref[...]
    a = lax.bitcast_convert_type(v << 16, _f32)
    b = lax.bitcast_convert_type(v & jnp.int32(-65536), _f32)
    ea = ea_ref[...]
    ta = jnp.maximum(
        a + jnp.dot(ea, wea_ref[...], preferred_element_type=_f32)
        + bea_ref[...], 0.0)
    tb = jnp.maximum(
        b + jnp.dot(ea, web_ref[...], preferred_element_type=_f32)
        + beb_ref[...], 0.0)
    m_ref[...] = (jnp.dot(ta, w1a_ref[...], preferred_element_type=_f32)
                  + jnp.dot(tb, w1b_ref[...], preferred_element_type=_f32))


def _edge0(xg, ea, wea, web, bea, beb, w1a, w1b):
    half = pl.BlockSpec((ED, IN // 2), lambda i: (0, 0))
    vec = pl.BlockSpec((1, IN // 2), lambda i: (0, 0))
    mat = pl.BlockSpec((IN // 2, HID), lambda i: (0, 0))
    return pl.pallas_call(
        _edge0_body,
        grid=(EP // BE1,),
        in_specs=[
            pl.BlockSpec((BE1, IN // 2), lambda i: (i, 0)),
            pl.BlockSpec((BE1, ED), lambda i: (i, 0)),
            half, half, vec, vec, mat, mat,
        ],
        out_specs=pl.BlockSpec((BE1, HID), lambda i: (i, 0)),
        out_shape=jax.ShapeDtypeStruct((EP, HID), _f32),
    )(xg, ea, wea, web, bea, beb, w1a, w1b)


def _edge_body(hg_ref, ea_ref, we_ref, be_ref, m_ref):
    emb = jnp.dot(ea_ref[...], we_ref[...], preferred_element_type=_f32)
    m_ref[...] = jnp.maximum(hg_ref[...] + emb + be_ref[...], 0.0)


def _edge(hg, ea, we, be_):
    return pl.pallas_call(
        _edge_body,
        grid=(EP // BE4,),
        in_specs=[
            pl.BlockSpec((BE4, HID), lambda i: (i, 0)),
            pl.BlockSpec((BE4, ED), lambda i: (i, 0)),
            pl.BlockSpec((ED, HID), lambda i: (0, 0)),
            pl.BlockSpec((1, HID), lambda i: (0, 0)),
        ],
        out_specs=pl.BlockSpec((BE4, HID), lambda i: (i, 0)),
        out_shape=jax.ShapeDtypeStruct((EP, HID), _f32),
    )(hg, ea, we, be_)


def _xw1_esm_body(x_ref, w1_ref, batch_ref, pos_ref, center_ref,
                  xw_ref, esm_ref):
    i = pl.program_id(0)
    xb = x_ref[...]
    xw_ref[...] = jnp.dot(xb, w1_ref[...], preferred_element_type=_f32)
    sel = _sel_block(batch_ref, pos_ref, center_ref)

    @pl.when(i == 0)
    def _():
        esm_ref[...] = jnp.zeros_like(esm_ref)

    esm_ref[...] += jnp.dot(sel, xb, preferred_element_type=_f32)


def _xw1_esm(x, w1, batch3, pos3, center2d):
    return pl.pallas_call(
        _xw1_esm_body,
        grid=(N // BN,),
        in_specs=[
            pl.BlockSpec((BN, IN), lambda i: (i, 0)),
            pl.BlockSpec((IN, HID), lambda i: (0, 0)),
            pl.BlockSpec((1, 1, BN), lambda i: (i, 0, 0)),
            pl.BlockSpec((1, 1, BN), lambda i: (i, 0, 0)),
            pl.BlockSpec((B, 128), lambda i: (0, 0)),
        ],
        out_specs=[
            pl.BlockSpec((BN, HID), lambda i: (i, 0)),
            pl.BlockSpec((B, IN), lambda i: (0, 0)),
        ],
        out_shape=[
            jax.ShapeDtypeStruct((N, HID), _f32),
            jax.ShapeDtypeStruct((B, IN), _f32),
        ],
    )(x, w1, batch3, pos3, center2d)


def _mlp_ln_tail(u, w2_ref, b2_ref, g_ref, bb_ref, out_ref):
    v = jnp.dot(u, w2_ref[...], preferred_element_type=_f32) + b2_ref[...]
    r = jnp.maximum(v, 0.0)
    mu = jnp.mean(r, axis=1, keepdims=True)
    var = jnp.mean((r - mu) * (r - mu), axis=1, keepdims=True)
    out_ref[...] = (r - mu) * lax.rsqrt(var + 1e-5) * g_ref[...] + bb_ref[...]


def _node0_body(xw_ref, agg_ref, b1_ref, w2_ref, b2_ref, g_ref, bb_ref,
                out_ref):
    u = jnp.maximum(xw_ref[...] + agg_ref[...] + b1_ref[...], 0.0)
    _mlp_ln_tail(u, w2_ref, b2_ref, g_ref, bb_ref, out_ref)


def _node0(xw, agg, b1, w2, b2, g, bb):
    vec = pl.BlockSpec((1, HID), lambda i: (0, 0))
    return pl.pallas_call(
        _node0_body,
        grid=(N // BN,),
        in_specs=[
            pl.BlockSpec((BN, HID), lambda i: (i, 0)),
            pl.BlockSpec((BN, HID), lambda i: (i, 0)),
            vec, pl.BlockSpec((HID, HID), lambda i: (0, 0)), vec, vec, vec,
        ],
        out_specs=pl.BlockSpec((BN, HID), lambda i: (i, 0)),
        out_shape=jax.ShapeDtypeStruct((N, HID), _f32),
    )(xw, agg, b1, w2, b2, g, bb)


def _node_body(h_ref, agg_ref, w1_ref, b1_ref, w2_ref, b2_ref, g_ref, bb_ref,
               out_ref):
    z = h_ref[...] + agg_ref[...]
    u = jnp.maximum(
        jnp.dot(z, w1_ref[...], preferred_element_type=_f32) + b1_ref[...],
        0.0)
    _mlp_ln_tail(u, w2_ref, b2_ref, g_ref, bb_ref, out_ref)


def _node(h, agg, w1, b1, w2, b2, g, bb):
    vec = pl.BlockSpec((1, HID), lambda i: (0, 0))
    mat = pl.BlockSpec((HID, HID), lambda i: (0, 0))
    return pl.pallas_call(
        _node_body,
        grid=(N // BN,),
        in_specs=[
            pl.BlockSpec((BN, HID), lambda i: (i, 0)),
            pl.BlockSpec((BN, HID), lambda i: (i, 0)),
            mat, vec, mat, vec, vec, vec,
        ],
        out_specs=pl.BlockSpec((BN, HID), lambda i: (i, 0)),
        out_shape=jax.ShapeDtypeStruct((N, HID), _f32),
    )(h, agg, w1, b1, w2, b2, g, bb)


def _head_body(h_ref, batch_ref, pos_ref, center_ref, esm_ref,
               wg_ref, we_ref, bf1_ref, wf2_ref, bf2_ref,
               out_ref, acc_ref):
    i = pl.program_id(0)
    sel = _sel_block(batch_ref, pos_ref, center_ref)

    @pl.when(i == 0)
    def _():
        acc_ref[...] = jnp.zeros_like(acc_ref)

    acc_ref[...] += jnp.dot(sel, h_ref[...], preferred_element_type=_f32)

    @pl.when(i == N // BN - 1)
    def _():
        g = (jnp.dot(acc_ref[...], wg_ref[...], preferred_element_type=_f32)
             + jnp.dot(esm_ref[...], we_ref[...], preferred_element_type=_f32)
             + bf1_ref[...])
        r = jnp.maximum(g, 0.0)
        out_ref[...] = (jnp.dot(r, wf2_ref[...], preferred_element_type=_f32)
                        + bf2_ref[...])


def _head(h3, batch3, pos3, center2d, esm, wf1g, wf1e, bf1, wf2p, bf2p):
    return pl.pallas_call(
        _head_body,
        grid=(N // BN,),
        in_specs=[
            pl.BlockSpec((BN, HID), lambda i: (i, 0)),
            pl.BlockSpec((1, 1, BN), lambda i: (i, 0, 0)),
            pl.BlockSpec((1, 1, BN), lambda i: (i, 0, 0)),
            pl.BlockSpec((B, 128), lambda i: (0, 0)),
            pl.BlockSpec((B, IN), lambda i: (0, 0)),
            pl.BlockSpec((HID, HID), lambda i: (0, 0)),
            pl.BlockSpec((IN, HID), lambda i: (0, 0)),
            pl.BlockSpec((1, HID), lambda i: (0, 0)),
            pl.BlockSpec((HID, 128), lambda i: (0, 0)),
            pl.BlockSpec((1, 128), lambda i: (0, 0)),
        ],
        out_specs=pl.BlockSpec((B, 128), lambda i: (0, 0)),
        out_shape=jax.ShapeDtypeStruct((B, 128), _f32),
        scratch_shapes=[pltpu.VMEM((B, HID), _f32)],
    )(h3, batch3, pos3, center2d, esm, wf1g, wf1e, bf1, wf2p, bf2p)


def kernel(x, edge_index, edge_attr, batch, pos_idx, center_idx, params):
    pad = EP - E
    src = edge_index[0]
    dst = edge_index[1]
    src_pad = jnp.concatenate([src, jnp.zeros((pad,), jnp.int32)])
    src3x = src_pad.reshape(32, EP // 32 // GW, GW)
    src3h = src_pad.reshape(32, EP // 32 // GWH, GWH)
    dst2 = jnp.concatenate(
        [dst, N + (jnp.arange(pad, dtype=jnp.int32) & 7)]
    ).reshape(16, EP // 16 // SW, SW)
    zrows = jnp.zeros((ZR, HC), _f32)
    ea_pad = jnp.concatenate([edge_attr, jnp.zeros((pad, ED), _f32)])
    batch3 = batch.reshape(N // BN, 1, BN)
    pos3 = pos_idx.reshape(N // BN, 1, BN)
    center2d = jnp.broadcast_to(center_idx[:, None], (B, 128))

    L = params['layers']
    r1 = lambda a: a.reshape(1, -1)
    _gather_x, _gather_h, _scatter_add = _sc_kernels()

    H2 = IN // 2
    xi = lax.bitcast_convert_type(x, jnp.int32) + jnp.int32(0x8000)
    xpk = (lax.shift_right_logical(xi[:, :H2], 16)
           | (xi[:, H2:] & jnp.int32(-65536)))
    xg = _gather_x(xpk, src3x)
    We0, be0, W10 = L[0]['We'], L[0]['be'], L[0]['W1']
    m0 = _edge0(xg, ea_pad, We0[:, :H2], We0[:, H2:],
                r1(be0[:H2]), r1(be0[H2:]), W10[:H2], W10[H2:])
    agg = _scatter_add(m0, dst2, zrows)[:N]
    xw, esm = _xw1_esm(x, L[0]['W1'], batch3, pos3, center2d)
    h = _node0(xw, agg, r1(L[0]['b1']), L[0]['W2'], r1(L[0]['b2']),
               r1(L[0]['gamma']), r1(L[0]['beta']))

    for p in L[1:]:
        hg = _gather_h(h, src3h)
        msg = _edge(hg, ea_pad, p['We'], r1(p['be']))
        agg = _scatter_add(msg, dst2, zrows)[:N]
        h = _node(h, agg, p['W1'], r1(p['b1']), p['W2'], r1(p['b2']),
                  r1(p['gamma']), r1(p['beta']))

    wf2p = jnp.pad(params['Wf2'], ((0, 0), (0, 126)))
    bf2p = jnp.pad(params['bf2'], (0, 126)).reshape(1, 128)
    out = _head(h, batch3, pos3, center2d, esm,
                params['Wf1'][:HID], params['Wf1'][HID:],
                r1(params['bf1']), wf2p, bf2p)
    return out[:, :2]


# speedup vs baseline: 2.4497x; 1.0216x over previous
import functools

import jax
import jax.numpy as jnp
from jax import lax
from jax.experimental import pallas as pl
from jax.experimental.pallas import tpu as pltpu
from jax.experimental.pallas import tpu_sc as plsc

N = 10000
E = 60000
B = 64
IN = 1280
ED = 16
HID = 256

EP = 61440
GW = 96
GWH = 128
SW = 128
BN = 400
BE1 = 512
BE4 = 1024

_f32 = jnp.float32


def _make_gather(d, mesh, dtype=_f32, gw=GW):
    chunk = EP // 32
    nwin = chunk // gw

    @functools.partial(
        pl.kernel,
        out_type=jax.ShapeDtypeStruct((EP, d), dtype),
        mesh=mesh,
        scratch_types=[
            pltpu.VMEM((nwin, gw), jnp.int32),
            pltpu.VMEM((gw, d), dtype),
            pltpu.SemaphoreType.DMA,
        ],
    )
    def gk(tbl_hbm, src3_hbm, out_hbm, idx_v, rows_v, sem):
        c = lax.axis_index("c")
        s = lax.axis_index("s")
        wid = s * 2 + c
        ebase = wid * chunk
        pltpu.sync_copy(src3_hbm.at[wid], idx_v)

        def body(g, carry):
            pltpu.async_copy(tbl_hbm.at[idx_v.at[g]], rows_v, sem).wait()
            pltpu.sync_copy(rows_v, out_hbm.at[pl.ds(ebase + g * gw, gw)])
            return carry

        lax.fori_loop(0, nwin, body, 0)

    return gk


NROW = 10112
ZR = NROW // 16
HC = HID // 2


def _make_scatter(mesh):
    nwin = EP // 16 // SW

    @functools.partial(
        pl.kernel,
        out_type=jax.ShapeDtypeStruct((NROW, HID), _f32),
        mesh=mesh,
        scratch_types=[
            pltpu.VMEM((nwin, SW), jnp.int32),
            pltpu.VMEM((SW, HC), _f32),
            pltpu.VMEM_SHARED((NROW, HC), _f32),
        ],
    )
    def _scatter_kernel(msg_hbm, dst3_hbm, zero_hbm, agg_hbm,
                        idxbuf, updbuf, acc):
        c = lax.axis_index("c")
        s = lax.axis_index("s")
        ebase = s * (EP // 16)
        col = c * HC
        pltpu.sync_copy(dst3_hbm.at[s], idxbuf)
        pltpu.sync_copy(zero_hbm, acc.at[pl.ds(s * ZR, ZR)])
        plsc.subcore_barrier()

        def body(g, carry):
            pltpu.sync_copy(
                msg_hbm.at[pl.ds(ebase + g * SW, SW), pl.ds(col, HC)],
                updbuf)
            pltpu.sync_copy(updbuf, acc.at[idxbuf.at[g]], add=True)
            return carry

        lax.fori_loop(0, nwin, body, 0)
        plsc.subcore_barrier()
        pltpu.sync_copy(
            acc.at[pl.ds(s * ZR, ZR)],
            agg_hbm.at[pl.ds(s * ZR, ZR), pl.ds(col, HC)])

    return _scatter_kernel


@functools.lru_cache(maxsize=1)
def _sc_kernels():
    mesh = plsc.VectorSubcoreMesh(core_axis_name="c", subcore_axis_name="s")
    return (_make_gather(IN // 2, mesh, jnp.int32, GW),
            _make_gather(HID, mesh, _f32, GWH),
            _make_scatter(mesh))


def _sel_block(batch_ref, pos_ref, center_ref):
    b = batch_ref[0, 0, :][None, :]
    p = pos_ref[0, 0, :][None, :]
    ci = center_ref[:, 0:1]
    ii = lax.broadcasted_iota(jnp.int32, (B, BN), 0)
    return ((b == ii) & (p == ci)).astype(_f32)


def _edge0_body(xg_ref, ea_ref, wea_ref, web_ref, bea_ref, beb_ref,
                w1a_ref, w1b_ref, m_ref):
    v = xg_ref[...]
    a = lax.bitcast_convert_type(v << 16, _f32)
    b = lax.bitcast_convert_type(v & jnp.int32(-65536), _f32)
    ea = ea_ref[...]
    ta = jnp.maximum(
        a + jnp.dot(ea, wea_ref[...], preferred_element_type=_f32)
        + bea_ref[...], 0.0)
    tb = jnp.maximum(
        b + jnp.dot(ea, web_ref[...], preferred_element_type=_f32)
        + beb_ref[...], 0.0)
    m_ref[...] = (jnp.dot(ta, w1a_ref[...], preferred_element_type=_f32)
                  + jnp.dot(tb, w1b_ref[...], preferred_element_type=_f32))


def _edge0(xg, ea, wea, web, bea, beb, w1a, w1b):
    half = pl.BlockSpec((ED, IN // 2), lambda i: (0, 0))
    vec = pl.BlockSpec((1, IN // 2), lambda i: (0, 0))
    mat = pl.BlockSpec((IN // 2, HID), lambda i: (0, 0))
    return pl.pallas_call(
        _edge0_body,
        grid=(EP // BE1,),
        in_specs=[
            pl.BlockSpec((BE1, IN // 2), lambda i: (i, 0)),
            pl.BlockSpec((BE1, ED), lambda i: (i, 0)),
            half, half, vec, vec, mat, mat,
        ],
        out_specs=pl.BlockSpec((BE1, HID), lambda i: (i, 0)),
        out_shape=jax.ShapeDtypeStruct((EP, HID), _f32),
    )(xg, ea, wea, web, bea, beb, w1a, w1b)


def _edge_body(hg_ref, ea_ref, we_ref, be_ref, m_ref):
    emb = jnp.dot(ea_ref[...], we_ref[...], preferred_element_type=_f32)
    m_ref[...] = jnp.maximum(hg_ref[...] + emb + be_ref[...], 0.0)


def _edge(hg, ea, we, be_):
    return pl.pallas_call(
        _edge_body,
        grid=(EP // BE4,),
        in_specs=[
            pl.BlockSpec((BE4, HID), lambda i: (i, 0)),
            pl.BlockSpec((BE4, ED), lambda i: (i, 0)),
            pl.BlockSpec((ED, HID), lambda i: (0, 0)),
            pl.BlockSpec((1, HID), lambda i: (0, 0)),
        ],
        out_specs=pl.BlockSpec((BE4, HID), lambda i: (i, 0)),
        out_shape=jax.ShapeDtypeStruct((EP, HID), _f32),
    )(hg, ea, we, be_)


def _xw1_esm_body(x_ref, w1_ref, batch_ref, pos_ref, center_ref,
                  xw_ref, esm_ref):
    i = pl.program_id(0)
    xb = x_ref[...]
    xw_ref[...] = jnp.dot(xb, w1_ref[...], preferred_element_type=_f32)
    sel = _sel_block(batch_ref, pos_ref, center_ref)

    @pl.when(i == 0)
    def _():
        esm_ref[...] = jnp.zeros_like(esm_ref)

    esm_ref[...] += jnp.dot(sel, xb, preferred_element_type=_f32)


def _xw1_esm(x, w1, batch3, pos3, center2d):
    return pl.pallas_call(
        _xw1_esm_body,
        grid=(N // BN,),
        in_specs=[
            pl.BlockSpec((BN, IN), lambda i: (i, 0)),
            pl.BlockSpec((IN, HID), lambda i: (0, 0)),
            pl.BlockSpec((1, 1, BN), lambda i: (i, 0, 0)),
            pl.BlockSpec((1, 1, BN), lambda i: (i, 0, 0)),
            pl.BlockSpec((B, 128), lambda i: (0, 0)),
        ],
        out_specs=[
            pl.BlockSpec((BN, HID), lambda i: (i, 0)),
            pl.BlockSpec((B, IN), lambda i: (0, 0)),
        ],
        out_shape=[
            jax.ShapeDtypeStruct((N, HID), _f32),
            jax.ShapeDtypeStruct((B, IN), _f32),
        ],
    )(x, w1, batch3, pos3, center2d)


def _mlp_ln_tail(u, w2_ref, b2_ref, g_ref, bb_ref, out_ref):
    v = jnp.dot(u, w2_ref[...], preferred_element_type=_f32) + b2_ref[...]
    r = jnp.maximum(v, 0.0)
    mu = jnp.mean(r, axis=1, keepdims=True)
    var = jnp.mean((r - mu) * (r - mu), axis=1, keepdims=True)
    out_ref[...] = (r - mu) * lax.rsqrt(var + 1e-5) * g_ref[...] + bb_ref[...]


def _node0_body(xw_ref, agg_ref, b1_ref, w2_ref, b2_ref, g_ref, bb_ref,
                out_ref):
    u = jnp.maximum(xw_ref[...] + agg_ref[...] + b1_ref[...], 0.0)
    _mlp_ln_tail(u, w2_ref, b2_ref, g_ref, bb_ref, out_ref)


def _node0(xw, agg, b1, w2, b2, g, bb):
    vec = pl.BlockSpec((1, HID), lambda i: (0, 0))
    return pl.pallas_call(
        _node0_body,
        grid=(N // BN,),
        in_specs=[
            pl.BlockSpec((BN, HID), lambda i: (i, 0)),
            pl.BlockSpec((BN, HID), lambda i: (i, 0)),
            vec, pl.BlockSpec((HID, HID), lambda i: (0, 0)), vec, vec, vec,
        ],
        out_specs=pl.BlockSpec((BN, HID), lambda i: (i, 0)),
        out_shape=jax.ShapeDtypeStruct((N, HID), _f32),
    )(xw, agg, b1, w2, b2, g, bb)


def _node_body(h_ref, agg_ref, w1_ref, b1_ref, w2_ref, b2_ref, g_ref, bb_ref,
               out_ref):
    z = h_ref[...] + agg_ref[...]
    u = jnp.maximum(
        jnp.dot(z, w1_ref[...], preferred_element_type=_f32) + b1_ref[...],
        0.0)
    _mlp_ln_tail(u, w2_ref, b2_ref, g_ref, bb_ref, out_ref)


def _node(h, agg, w1, b1, w2, b2, g, bb):
    vec = pl.BlockSpec((1, HID), lambda i: (0, 0))
    mat = pl.BlockSpec((HID, HID), lambda i: (0, 0))
    return pl.pallas_call(
        _node_body,
        grid=(N // BN,),
        in_specs=[
            pl.BlockSpec((BN, HID), lambda i: (i, 0)),
            pl.BlockSpec((BN, HID), lambda i: (i, 0)),
            mat, vec, mat, vec, vec, vec,
        ],
        out_specs=pl.BlockSpec((BN, HID), lambda i: (i, 0)),
        out_shape=jax.ShapeDtypeStruct((N, HID), _f32),
    )(h, agg, w1, b1, w2, b2, g, bb)


def _head_body(h_ref, batch_ref, pos_ref, center_ref, esm_ref,
               wg_ref, we_ref, bf1_ref, wf2_ref, bf2_ref,
               out_ref, acc_ref):
    i = pl.program_id(0)
    sel = _sel_block(batch_ref, pos_ref, center_ref)

    @pl.when(i == 0)
    def _():
        acc_ref[...] = jnp.zeros_like(acc_ref)

    acc_ref[...] += jnp.dot(sel, h_ref[...], preferred_element_type=_f32)

    @pl.when(i == N // BN - 1)
    def _():
        g = (jnp.dot(acc_ref[...], wg_ref[...], preferred_element_type=_f32)
             + jnp.dot(esm_ref[...], we_ref[...], preferred_element_type=_f32)
             + bf1_ref[...])
        r = jnp.maximum(g, 0.0)
        out_ref[...] = (jnp.dot(r, wf2_ref[...], preferred_element_type=_f32)
                        + bf2_ref[...])


def _head(h3, batch3, pos3, center2d, esm, wf1g, wf1e, bf1, wf2p, bf2p):
    return pl.pallas_call(
        _head_body,
        grid=(N // BN,),
        in_specs=[
            pl.BlockSpec((BN, HID), lambda i: (i, 0)),
            pl.BlockSpec((1, 1, BN), lambda i: (i, 0, 0)),
            pl.BlockSpec((1, 1, BN), lambda i: (i, 0, 0)),
            pl.BlockSpec((B, 128), lambda i: (0, 0)),
            pl.BlockSpec((B, IN), lambda i: (0, 0)),
            pl.BlockSpec((HID, HID), lambda i: (0, 0)),
            pl.BlockSpec((IN, HID), lambda i: (0, 0)),
            pl.BlockSpec((1, HID), lambda i: (0, 0)),
            pl.BlockSpec((HID, 128), lambda i: (0, 0)),
            pl.BlockSpec((1, 128), lambda i: (0, 0)),
        ],
        out_specs=pl.BlockSpec((B, 128), lambda i: (0, 0)),
        out_shape=jax.ShapeDtypeStruct((B, 128), _f32),
        scratch_shapes=[pltpu.VMEM((B, HID), _f32)],
    )(h3, batch3, pos3, center2d, esm, wf1g, wf1e, bf1, wf2p, bf2p)


def kernel(x, edge_index, edge_attr, batch, pos_idx, center_idx, params):
    pad = EP - E
    src = edge_index[0]
    dst = edge_index[1]
    src_pad = jnp.concatenate([src, jnp.zeros((pad,), jnp.int32)])
    src3x = src_pad.reshape(32, EP // 32 // GW, GW)
    src3h = src_pad.reshape(32, EP // 32 // GWH, GWH)
    dst2 = jnp.concatenate(
        [dst, N + (jnp.arange(pad, dtype=jnp.int32) & 7)]
    ).reshape(16, EP // 16 // SW, SW)
    zrows = jnp.zeros((ZR, HC), _f32)
    ea_pad = jnp.concatenate([edge_attr, jnp.zeros((pad, ED), _f32)])
    batch3 = batch.reshape(N // BN, 1, BN)
    pos3 = pos_idx.reshape(N // BN, 1, BN)
    center2d = jnp.broadcast_to(center_idx[:, None], (B, 128))

    L = params['layers']
    r1 = lambda a: a.reshape(1, -1)
    _gather_x, _gather_h, _scatter_add = _sc_kernels()

    H2 = IN // 2
    xi = lax.bitcast_convert_type(x, jnp.int32) + jnp.int32(0x8000)
    xpk = (lax.shift_right_logical(xi[:, :H2], 16)
           | (xi[:, H2:] & jnp.int32(-65536)))
    xg = _gather_x(xpk, src3x)
    We0, be0, W10 = L[0]['We'], L[0]['be'], L[0]['W1']
    m0 = _edge0(xg, ea_pad, We0[:, :H2], We0[:, H2:],
                r1(be0[:H2]), r1(be0[H2:]), W10[:H2], W10[H2:])
    agg = _scatter_add(m0, dst2, zrows)[:N]
    xw, esm = _xw1_esm(x, L[0]['W1'], batch3, pos3, center2d)
    h = _node0(xw, agg, r1(L[0]['b1']), L[0]['W2'], r1(L[0]['b2']),
               r1(L[0]['gamma']), r1(L[0]['beta']))

    for p in L[1:]:
        hg = _gather_h(h, src3h)
        msg = _edge(hg, ea_pad, p['We'], r1(p['be']))
        agg = _scatter_add(msg, dst2, zrows)[:N]
        h = _node(h, agg, p['W1'], r1(p['b1']), p['W2'], r1(p['b2']),
                  r1(p['gamma']), r1(p['beta']))

    wf2p = jnp.pad(params['Wf2'], ((0, 0), (0, 126)))
    bf2p = jnp.pad(params['bf2'], (0, 126)).reshape(1, 128)
    out = _head(h, batch3, pos3, center2d, esm,
                params['Wf1'][:HID], params['Wf1'][HID:],
                r1(params['bf1']), wf2p, bf2p)
    return out[:, :2]


# speedup vs baseline: 2.5975x; 1.0603x over previous
import functools

import jax
import jax.numpy as jnp
from jax import lax
from jax.experimental import pallas as pl
from jax.experimental.pallas import tpu as pltpu
from jax.experimental.pallas import tpu_sc as plsc

N = 10000
E = 60000
B = 64
IN = 1280
ED = 16
HID = 256

EP = 61440
EPH = EP // 2
GW = 96
SW = 128
BN = 400
BE1 = 512
BE4 = 1024

_f32 = jnp.float32


def _make_gather(d, mesh, dtype=_f32, gw=GW):
    chunk = EPH // 32
    nwin = chunk // gw

    @functools.partial(
        pl.kernel,
        out_type=jax.ShapeDtypeStruct((EPH, d), dtype),
        mesh=mesh,
        scratch_types=[
            pltpu.VMEM((nwin, gw), jnp.int32),
            pltpu.VMEM((gw, d), dtype),
            pltpu.SemaphoreType.DMA,
        ],
    )
    def gk(tbl_hbm, src3_hbm, out_hbm, idx_v, rows_v, sem):
        c = lax.axis_index("c")
        s = lax.axis_index("s")
        wid = s * 2 + c
        ebase = wid * chunk
        pltpu.sync_copy(src3_hbm.at[wid], idx_v)

        def body(g, carry):
            pltpu.async_copy(tbl_hbm.at[idx_v.at[g]], rows_v, sem).wait()
            pltpu.sync_copy(rows_v, out_hbm.at[pl.ds(ebase + g * gw, gw)])
            return carry

        lax.fori_loop(0, nwin, body, 0)

    return gk


NROW = 10112
ZR = NROW // 16
HC = HID // 2


def _make_scatter(mesh):
    nwin = EPH // 16 // SW

    @functools.partial(
        pl.kernel,
        out_type=jax.ShapeDtypeStruct((NROW, HID), _f32),
        mesh=mesh,
        scratch_types=[
            pltpu.VMEM((nwin, SW), jnp.int32),
            pltpu.VMEM((nwin, SW), jnp.int32),
            pltpu.VMEM((SW, HC), _f32),
            pltpu.VMEM_SHARED((NROW, HC), _f32),
        ],
    )
    def _scatter_kernel(msga_hbm, msgb_hbm, dsta_hbm, dstb_hbm, zero_hbm,
                        agg_hbm, idxa, idxb, updbuf, acc):
        c = lax.axis_index("c")
        s = lax.axis_index("s")
        ebase = s * (EPH // 16)
        col = c * HC
        pltpu.sync_copy(dsta_hbm.at[s], idxa)
        pltpu.sync_copy(dstb_hbm.at[s], idxb)
        pltpu.sync_copy(zero_hbm, acc.at[pl.ds(s * ZR, ZR)])
        plsc.subcore_barrier()

        def slab(msg_hbm, idxbuf):
            def body(g, carry):
                pltpu.sync_copy(
                    msg_hbm.at[pl.ds(ebase + g * SW, SW), pl.ds(col, HC)],
                    updbuf)
                pltpu.sync_copy(updbuf, acc.at[idxbuf.at[g]], add=True)
                return carry
            lax.fori_loop(0, nwin, body, 0)

        slab(msga_hbm, idxa)
        slab(msgb_hbm, idxb)
        plsc.subcore_barrier()
        pltpu.sync_copy(
            acc.at[pl.ds(s * ZR, ZR)],
            agg_hbm.at[pl.ds(s * ZR, ZR), pl.ds(col, HC)])

    return _scatter_kernel


@functools.lru_cache(maxsize=1)
def _sc_kernels():
    mesh = plsc.VectorSubcoreMesh(core_axis_name="c", subcore_axis_name="s")
    return (_make_gather(IN // 2, mesh, jnp.int32, GW),
            _make_gather(HID, mesh, _f32, GW),
            _make_scatter(mesh))


def _sel_block(batch_ref, pos_ref, center_ref):
    b = batch_ref[0, 0, :][None, :]
    p = pos_ref[0, 0, :][None, :]
    ci = center_ref[:, 0:1]
    ii = lax.broadcasted_iota(jnp.int32, (B, BN), 0)
    return ((b == ii) & (p == ci)).astype(_f32)


def _edge0_body(xg_ref, ea_ref, wea_ref, web_ref, bea_ref, beb_ref,
                w1a_ref, w1b_ref, m_ref):
    v = xg_ref[...]
    a = lax.bitcast_convert_type(v << 16, _f32)
    b = lax.bitcast_convert_type(v & jnp.int32(-65536), _f32)
    ea = ea_ref[...]
    ta = jnp.maximum(
        a + jnp.dot(ea, wea_ref[...], preferred_element_type=_f32)
        + bea_ref[...], 0.0)
    tb = jnp.maximum(
        b + jnp.dot(ea, web_ref[...], preferred_element_type=_f32)
        + beb_ref[...], 0.0)
    m_ref[...] = (jnp.dot(ta, w1a_ref[...], preferred_element_type=_f32)
                  + jnp.dot(tb, w1b_ref[...], preferred_element_type=_f32))


def _edge0(xg, ea, wea, web, bea, beb, w1a, w1b):
    half = pl.BlockSpec((ED, IN // 2), lambda i: (0, 0))
    vec = pl.BlockSpec((1, IN // 2), lambda i: (0, 0))
    mat = pl.BlockSpec((IN // 2, HID), lambda i: (0, 0))
    ne = xg.shape[0]
    return pl.pallas_call(
        _edge0_body,
        grid=(ne // BE1,),
        in_specs=[
            pl.BlockSpec((BE1, IN // 2), lambda i: (i, 0)),
            pl.BlockSpec((BE1, ED), lambda i: (i, 0)),
            half, half, vec, vec, mat, mat,
        ],
        out_specs=pl.BlockSpec((BE1, HID), lambda i: (i, 0)),
        out_shape=jax.ShapeDtypeStruct((ne, HID), _f32),
    )(xg, ea, wea, web, bea, beb, w1a, w1b)


def _edge_body(hg_ref, ea_ref, we_ref, be_ref, m_ref):
    emb = jnp.dot(ea_ref[...], we_ref[...], preferred_element_type=_f32)
    m_ref[...] = jnp.maximum(hg_ref[...] + emb + be_ref[...], 0.0)


def _edge(hg, ea, we, be_):
    ne = hg.shape[0]
    return pl.pallas_call(
        _edge_body,
        grid=(ne // BE4,),
        in_specs=[
            pl.BlockSpec((BE4, HID), lambda i: (i, 0)),
            pl.BlockSpec((BE4, ED), lambda i: (i, 0)),
            pl.BlockSpec((ED, HID), lambda i: (0, 0)),
            pl.BlockSpec((1, HID), lambda i: (0, 0)),
        ],
        out_specs=pl.BlockSpec((BE4, HID), lambda i: (i, 0)),
        out_shape=jax.ShapeDtypeStruct((ne, HID), _f32),
    )(hg, ea, we, be_)


def _xw1_esm_body(x_ref, w1_ref, batch_ref, pos_ref, center_ref,
                  xw_ref, esm_ref):
    i = pl.program_id(0)
    xb = x_ref[...]
    xw_ref[...] = jnp.dot(xb, w1_ref[...], preferred_element_type=_f32)
    sel = _sel_block(batch_ref, pos_ref, center_ref)

    @pl.when(i == 0)
    def _():
        esm_ref[...] = jnp.zeros_like(esm_ref)

    esm_ref[...] += jnp.dot(sel, xb, preferred_element_type=_f32)


def _xw1_esm(x, w1, batch3, pos3, center2d):
    return pl.pallas_call(
        _xw1_esm_body,
        grid=(N // BN,),
        in_specs=[
            pl.BlockSpec((BN, IN), lambda i: (i, 0)),
            pl.BlockSpec((IN, HID), lambda i: (0, 0)),
            pl.BlockSpec((1, 1, BN), lambda i: (i, 0, 0)),
            pl.BlockSpec((1, 1, BN), lambda i: (i, 0, 0)),
            pl.BlockSpec((B, 128), lambda i: (0, 0)),
        ],
        out_specs=[
            pl.BlockSpec((BN, HID), lambda i: (i, 0)),
            pl.BlockSpec((B, IN), lambda i: (0, 0)),
        ],
        out_shape=[
            jax.ShapeDtypeStruct((N, HID), _f32),
            jax.ShapeDtypeStruct((B, IN), _f32),
        ],
    )(x, w1, batch3, pos3, center2d)


def _mlp_ln_tail(u, w2_ref, b2_ref, g_ref, bb_ref, out_ref):
    v = jnp.dot(u, w2_ref[...], preferred_element_type=_f32) + b2_ref[...]
    r = jnp.maximum(v, 0.0)
    mu = jnp.mean(r, axis=1, keepdims=True)
    var = jnp.mean((r - mu) * (r - mu), axis=1, keepdims=True)
    out_ref[...] = (r - mu) * lax.rsqrt(var + 1e-5) * g_ref[...] + bb_ref[...]


def _node0_body(xw_ref, agg_ref, b1_ref, w2_ref, b2_ref, g_ref, bb_ref,
                out_ref):
    u = jnp.maximum(xw_ref[...] + agg_ref[...] + b1_ref[...], 0.0)
    _mlp_ln_tail(u, w2_ref, b2_ref, g_ref, bb_ref, out_ref)


def _node0(xw, agg, b1, w2, b2, g, bb):
    vec = pl.BlockSpec((1, HID), lambda i: (0, 0))
    return pl.pallas_call(
        _node0_body,
        grid=(N // BN,),
        in_specs=[
            pl.BlockSpec((BN, HID), lambda i: (i, 0)),
            pl.BlockSpec((BN, HID), lambda i: (i, 0)),
            vec, pl.BlockSpec((HID, HID), lambda i: (0, 0)), vec, vec, vec,
        ],
        out_specs=pl.BlockSpec((BN, HID), lambda i: (i, 0)),
        out_shape=jax.ShapeDtypeStruct((N, HID), _f32),
    )(xw, agg, b1, w2, b2, g, bb)


def _node_body(h_ref, agg_ref, w1_ref, b1_ref, w2_ref, b2_ref, g_ref, bb_ref,
               out_ref):
    z = h_ref[...] + agg_ref[...]
    u = jnp.maximum(
        jnp.dot(z, w1_ref[...], preferred_element_type=_f32) + b1_ref[...],
        0.0)
    _mlp_ln_tail(u, w2_ref, b2_ref, g_ref, bb_ref, out_ref)


def _node(h, agg, w1, b1, w2, b2, g, bb):
    vec = pl.BlockSpec((1, HID), lambda i: (0, 0))
    mat = pl.BlockSpec((HID, HID), lambda i: (0, 0))
    return pl.pallas_call(
        _node_body,
        grid=(N // BN,),
        in_specs=[
            pl.BlockSpec((BN, HID), lambda i: (i, 0)),
            pl.BlockSpec((BN, HID), lambda i: (i, 0)),
            mat, vec, mat, vec, vec, vec,
        ],
        out_specs=pl.BlockSpec((BN, HID), lambda i: (i, 0)),
        out_shape=jax.ShapeDtypeStruct((N, HID), _f32),
    )(h, agg, w1, b1, w2, b2, g, bb)


def _head_body(h_ref, batch_ref, pos_ref, center_ref, esm_ref,
               wg_ref, we_ref, bf1_ref, wf2_ref, bf2_ref,
               out_ref, acc_ref):
    i = pl.program_id(0)
    sel = _sel_block(batch_ref, pos_ref, center_ref)

    @pl.when(i == 0)
    def _():
        acc_ref[...] = jnp.zeros_like(acc_ref)

    acc_ref[...] += jnp.dot(sel, h_ref[...], preferred_element_type=_f32)

    @pl.when(i == N // BN - 1)
    def _():
        g = (jnp.dot(acc_ref[...], wg_ref[...], preferred_element_type=_f32)
             + jnp.dot(esm_ref[...], we_ref[...], preferred_element_type=_f32)
             + bf1_ref[...])
        r = jnp.maximum(g, 0.0)
        out_ref[...] = (jnp.dot(r, wf2_ref[...], preferred_element_type=_f32)
                        + bf2_ref[...])


def _head(h3, batch3, pos3, center2d, esm, wf1g, wf1e, bf1, wf2p, bf2p):
    return pl.pallas_call(
        _head_body,
        grid=(N // BN,),
        in_specs=[
            pl.BlockSpec((BN, HID), lambda i: (i, 0)),
            pl.BlockSpec((1, 1, BN), lambda i: (i, 0, 0)),
            pl.BlockSpec((1, 1, BN), lambda i: (i, 0, 0)),
            pl.BlockSpec((B, 128), lambda i: (0, 0)),
            pl.BlockSpec((B, IN), lambda i: (0, 0)),
            pl.BlockSpec((HID, HID), lambda i: (0, 0)),
            pl.BlockSpec((IN, HID), lambda i: (0, 0)),
            pl.BlockSpec((1, HID), lambda i: (0, 0)),
            pl.BlockSpec((HID, 128), lambda i: (0, 0)),
            pl.BlockSpec((1, 128), lambda i: (0, 0)),
        ],
        out_specs=pl.BlockSpec((B, 128), lambda i: (0, 0)),
        out_shape=jax.ShapeDtypeStruct((B, 128), _f32),
        scratch_shapes=[pltpu.VMEM((B, HID), _f32)],
    )(h3, batch3, pos3, center2d, esm, wf1g, wf1e, bf1, wf2p, bf2p)


def kernel(x, edge_index, edge_attr, batch, pos_idx, center_idx, params):
    pad = EP - E
    src = edge_index[0]
    dst = edge_index[1]
    src_pad = jnp.concatenate([src, jnp.zeros((pad,), jnp.int32)])
    srcA = src_pad[:EPH].reshape(32, EPH // 32 // GW, GW)
    srcB = src_pad[EPH:].reshape(32, EPH // 32 // GW, GW)
    dst_pad = jnp.concatenate(
        [dst, N + (jnp.arange(pad, dtype=jnp.int32) & 7)])
    dstA = dst_pad[:EPH].reshape(16, EPH // 16 // SW, SW)
    dstB = dst_pad[EPH:].reshape(16, EPH // 16 // SW, SW)
    zrows = jnp.zeros((ZR, HC), _f32)
    ea_pad = jnp.concatenate([edge_attr, jnp.zeros((pad, ED), _f32)])
    eaA, eaB = ea_pad[:EPH], ea_pad[EPH:]
    batch3 = batch.reshape(N // BN, 1, BN)
    pos3 = pos_idx.reshape(N // BN, 1, BN)
    center2d = jnp.broadcast_to(center_idx[:, None], (B, 128))

    L = params['layers']
    r1 = lambda a: a.reshape(1, -1)
    _gather_x, _gather_h, _scatter_add = _sc_kernels()

    H2 = IN // 2
    xi = lax.bitcast_convert_type(x, jnp.int32) + jnp.int32(0x8000)
    xpk = (lax.shift_right_logical(xi[:, :H2], 16)
           | (xi[:, H2:] & jnp.int32(-65536)))
    We0, be0, W10 = L[0]['We'], L[0]['be'], L[0]['W1']
    ew0 = (We0[:, :H2], We0[:, H2:], r1(be0[:H2]), r1(be0[H2:]),
           W10[:H2], W10[H2:])
    xgA = _gather_x(xpk, srcA)
    m0A = _edge0(xgA, eaA, *ew0)
    xgB = _gather_x(xpk, srcB)
    m0B = _edge0(xgB, eaB, *ew0)
    agg = _scatter_add(m0A, m0B, dstA, dstB, zrows)[:N]
    xw, esm = _xw1_esm(x, L[0]['W1'], batch3, pos3, center2d)
    h = _node0(xw, agg, r1(L[0]['b1']), L[0]['W2'], r1(L[0]['b2']),
               r1(L[0]['gamma']), r1(L[0]['beta']))

    for p in L[1:]:
        hgA = _gather_h(h, srcA)
        mA = _edge(hgA, eaA, p['We'], r1(p['be']))
        hgB = _gather_h(h, srcB)
        mB = _edge(hgB, eaB, p['We'], r1(p['be']))
        agg = _scatter_add(mA, mB, dstA, dstB, zrows)[:N]
        h = _node(h, agg, p['W1'], r1(p['b1']), p['W2'], r1(p['b2']),
                  r1(p['gamma']), r1(p['beta']))

    wf2p = jnp.pad(params['Wf2'], ((0, 0), (0, 126)))
    bf2p = jnp.pad(params['bf2'], (0, 126)).reshape(1, 128)
    out = _head(h, batch3, pos3, center2d, esm,
                params['Wf1'][:HID], params['Wf1'][HID:],
                r1(params['bf1']), wf2p, bf2p)
    return out[:, :2]


# speedup vs baseline: 2.6097x; 1.0047x over previous
import functools

import jax
import jax.numpy as jnp
from jax import lax
from jax.experimental import pallas as pl
from jax.experimental.pallas import tpu as pltpu
from jax.experimental.pallas import tpu_sc as plsc

N = 10000
E = 60000
B = 64
IN = 1280
ED = 16
HID = 256

EP = 61440
EPH = EP // 2
GW = 120
SW = 128
BN = 400
BE1 = 512
BE4 = 1024

_f32 = jnp.float32


def _make_gather(d, mesh, dtype=_f32, gw=GW):
    chunk = EPH // 32
    nwin = chunk // gw

    @functools.partial(
        pl.kernel,
        out_type=jax.ShapeDtypeStruct((EPH, d), dtype),
        mesh=mesh,
        scratch_types=[
            pltpu.VMEM((nwin, gw), jnp.int32),
            pltpu.VMEM((gw, d), dtype),
            pltpu.SemaphoreType.DMA,
        ],
    )
    def gk(tbl_hbm, src3_hbm, out_hbm, idx_v, rows_v, sem):
        c = lax.axis_index("c")
        s = lax.axis_index("s")
        wid = s * 2 + c
        ebase = wid * chunk
        pltpu.sync_copy(src3_hbm.at[wid], idx_v)

        def body(g, carry):
            pltpu.async_copy(tbl_hbm.at[idx_v.at[g]], rows_v, sem).wait()
            pltpu.sync_copy(rows_v, out_hbm.at[pl.ds(ebase + g * gw, gw)])
            return carry

        lax.fori_loop(0, nwin, body, 0)

    return gk


NROW = 10112
ZR = NROW // 16
HC = HID // 2


def _make_scatter(mesh):
    nwin = EPH // 16 // SW

    @functools.partial(
        pl.kernel,
        out_type=jax.ShapeDtypeStruct((NROW, HID), _f32),
        mesh=mesh,
        scratch_types=[
            pltpu.VMEM((nwin, SW), jnp.int32),
            pltpu.VMEM((nwin, SW), jnp.int32),
            pltpu.VMEM((SW, HC), _f32),
            pltpu.VMEM_SHARED((NROW, HC), _f32),
        ],
    )
    def _scatter_kernel(msga_hbm, msgb_hbm, dsta_hbm, dstb_hbm, zero_hbm,
                        agg_hbm, idxa, idxb, updbuf, acc):
        c = lax.axis_index("c")
        s = lax.axis_index("s")
        ebase = s * (EPH // 16)
        col = c * HC
        pltpu.sync_copy(dsta_hbm.at[s], idxa)
        pltpu.sync_copy(dstb_hbm.at[s], idxb)
        pltpu.sync_copy(zero_hbm, acc.at[pl.ds(s * ZR, ZR)])
        plsc.subcore_barrier()

        def slab(msg_hbm, idxbuf):
            def body(g, carry):
                pltpu.sync_copy(
                    msg_hbm.at[pl.ds(ebase + g * SW, SW), pl.ds(col, HC)],
                    updbuf)
                pltpu.sync_copy(updbuf, acc.at[idxbuf.at[g]], add=True)
                return carry
            lax.fori_loop(0, nwin, body, 0)

        slab(msga_hbm, idxa)
        slab(msgb_hbm, idxb)
        plsc.subcore_barrier()
        pltpu.sync_copy(
            acc.at[pl.ds(s * ZR, ZR)],
            agg_hbm.at[pl.ds(s * ZR, ZR), pl.ds(col, HC)])

    return _scatter_kernel


@functools.lru_cache(maxsize=1)
def _sc_kernels():
    mesh = plsc.VectorSubcoreMesh(core_axis_name="c", subcore_axis_name="s")
    return (_make_gather(IN // 2, mesh, jnp.int32, GW),
            _make_gather(HID, mesh, _f32, GW),
            _make_scatter(mesh))


def _sel_block(batch_ref, pos_ref, center_ref):
    b = batch_ref[0, 0, :][None, :]
    p = pos_ref[0, 0, :][None, :]
    ci = center_ref[:, 0:1]
    ii = lax.broadcasted_iota(jnp.int32, (B, BN), 0)
    return ((b == ii) & (p == ci)).astype(_f32)


def _edge0_body(xg_ref, ea_ref, wea_ref, web_ref, bea_ref, beb_ref,
                w1a_ref, w1b_ref, m_ref):
    v = xg_ref[...]
    a = lax.bitcast_convert_type(v << 16, _f32)
    b = lax.bitcast_convert_type(v & jnp.int32(-65536), _f32)
    ea = ea_ref[...]
    ta = jnp.maximum(
        a + jnp.dot(ea, wea_ref[...], preferred_element_type=_f32)
        + bea_ref[...], 0.0)
    tb = jnp.maximum(
        b + jnp.dot(ea, web_ref[...], preferred_element_type=_f32)
        + beb_ref[...], 0.0)
    m_ref[...] = (jnp.dot(ta, w1a_ref[...], preferred_element_type=_f32)
                  + jnp.dot(tb, w1b_ref[...], preferred_element_type=_f32))


def _edge0(xg, ea, wea, web, bea, beb, w1a, w1b):
    half = pl.BlockSpec((ED, IN // 2), lambda i: (0, 0))
    vec = pl.BlockSpec((1, IN // 2), lambda i: (0, 0))
    mat = pl.BlockSpec((IN // 2, HID), lambda i: (0, 0))
    ne = xg.shape[0]
    return pl.pallas_call(
        _edge0_body,
        grid=(ne // BE1,),
        in_specs=[
            pl.BlockSpec((BE1, IN // 2), lambda i: (i, 0)),
            pl.BlockSpec((BE1, ED), lambda i: (i, 0)),
            half, half, vec, vec, mat, mat,
        ],
        out_specs=pl.BlockSpec((BE1, HID), lambda i: (i, 0)),
        out_shape=jax.ShapeDtypeStruct((ne, HID), _f32),
    )(xg, ea, wea, web, bea, beb, w1a, w1b)


def _edge_body(hg_ref, ea_ref, we_ref, be_ref, m_ref):
    emb = jnp.dot(ea_ref[...], we_ref[...], preferred_element_type=_f32)
    m_ref[...] = jnp.maximum(hg_ref[...] + emb + be_ref[...], 0.0)


def _edge(hg, ea, we, be_):
    ne = hg.shape[0]
    return pl.pallas_call(
        _edge_body,
        grid=(ne // BE4,),
        in_specs=[
            pl.BlockSpec((BE4, HID), lambda i: (i, 0)),
            pl.BlockSpec((BE4, ED), lambda i: (i, 0)),
            pl.BlockSpec((ED, HID), lambda i: (0, 0)),
            pl.BlockSpec((1, HID), lambda i: (0, 0)),
        ],
        out_specs=pl.BlockSpec((BE4, HID), lambda i: (i, 0)),
        out_shape=jax.ShapeDtypeStruct((ne, HID), _f32),
    )(hg, ea, we, be_)


def _xw1_esm_body(x_ref, w1_ref, batch_ref, pos_ref, center_ref,
                  xw_ref, esm_ref):
    i = pl.program_id(0)
    xb = x_ref[...]
    xw_ref[...] = jnp.dot(xb, w1_ref[...], preferred_element_type=_f32)
    sel = _sel_block(batch_ref, pos_ref, center_ref)

    @pl.when(i == 0)
    def _():
        esm_ref[...] = jnp.zeros_like(esm_ref)

    esm_ref[...] += jnp.dot(sel, xb, preferred_element_type=_f32)


def _xw1_esm(x, w1, batch3, pos3, center2d):
    return pl.pallas_call(
        _xw1_esm_body,
        grid=(N // BN,),
        in_specs=[
            pl.BlockSpec((BN, IN), lambda i: (i, 0)),
            pl.BlockSpec((IN, HID), lambda i: (0, 0)),
            pl.BlockSpec((1, 1, BN), lambda i: (i, 0, 0)),
            pl.BlockSpec((1, 1, BN), lambda i: (i, 0, 0)),
            pl.BlockSpec((B, 128), lambda i: (0, 0)),
        ],
        out_specs=[
            pl.BlockSpec((BN, HID), lambda i: (i, 0)),
            pl.BlockSpec((B, IN), lambda i: (0, 0)),
        ],
        out_shape=[
            jax.ShapeDtypeStruct((N, HID), _f32),
            jax.ShapeDtypeStruct((B, IN), _f32),
        ],
    )(x, w1, batch3, pos3, center2d)


def _mlp_ln_tail(u, w2_ref, b2_ref, g_ref, bb_ref, out_ref):
    v = jnp.dot(u, w2_ref[...], preferred_element_type=_f32) + b2_ref[...]
    r = jnp.maximum(v, 0.0)
    mu = jnp.mean(r, axis=1, keepdims=True)
    var = jnp.mean((r - mu) * (r - mu), axis=1, keepdims=True)
    out_ref[...] = (r - mu) * lax.rsqrt(var + 1e-5) * g_ref[...] + bb_ref[...]


def _node0_body(xw_ref, agg_ref, b1_ref, w2_ref, b2_ref, g_ref, bb_ref,
                out_ref):
    u = jnp.maximum(xw_ref[...] + agg_ref[...] + b1_ref[...], 0.0)
    _mlp_ln_tail(u, w2_ref, b2_ref, g_ref, bb_ref, out_ref)


def _node0(xw, agg, b1, w2, b2, g, bb):
    vec = pl.BlockSpec((1, HID), lambda i: (0, 0))
    return pl.pallas_call(
        _node0_body,
        grid=(N // BN,),
        in_specs=[
            pl.BlockSpec((BN, HID), lambda i: (i, 0)),
            pl.BlockSpec((BN, HID), lambda i: (i, 0)),
            vec, pl.BlockSpec((HID, HID), lambda i: (0, 0)), vec, vec, vec,
        ],
        out_specs=pl.BlockSpec((BN, HID), lambda i: (i, 0)),
        out_shape=jax.ShapeDtypeStruct((N, HID), _f32),
    )(xw, agg, b1, w2, b2, g, bb)


def _node_body(h_ref, agg_ref, w1_ref, b1_ref, w2_ref, b2_ref, g_ref, bb_ref,
               out_ref):
    z = h_ref[...] + agg_ref[...]
    u = jnp.maximum(
        jnp.dot(z, w1_ref[...], preferred_element_type=_f32) + b1_ref[...],
        0.0)
    _mlp_ln_tail(u, w2_ref, b2_ref, g_ref, bb_ref, out_ref)


def _node(h, agg, w1, b1, w2, b2, g, bb):
    vec = pl.BlockSpec((1, HID), lambda i: (0, 0))
    mat = pl.BlockSpec((HID, HID), lambda i: (0, 0))
    return pl.pallas_call(
        _node_body,
        grid=(N // BN,),
        in_specs=[
            pl.BlockSpec((BN, HID), lambda i: (i, 0)),
            pl.BlockSpec((BN, HID), lambda i: (i, 0)),
            mat, vec, mat, vec, vec, vec,
        ],
        out_specs=pl.BlockSpec((BN, HID), lambda i: (i, 0)),
        out_shape=jax.ShapeDtypeStruct((N, HID), _f32),
    )(h, agg, w1, b1, w2, b2, g, bb)


def _head_body(h_ref, batch_ref, pos_ref, center_ref, esm_ref,
               wg_ref, we_ref, bf1_ref, wf2_ref, bf2_ref,
               out_ref, acc_ref):
    i = pl.program_id(0)
    sel = _sel_block(batch_ref, pos_ref, center_ref)

    @pl.when(i == 0)
    def _():
        acc_ref[...] = jnp.zeros_like(acc_ref)

    acc_ref[...] += jnp.dot(sel, h_ref[...], preferred_element_type=_f32)

    @pl.when(i == N // BN - 1)
    def _():
        g = (jnp.dot(acc_ref[...], wg_ref[...], preferred_element_type=_f32)
             + jnp.dot(esm_ref[...], we_ref[...], preferred_element_type=_f32)
             + bf1_ref[...])
        r = jnp.maximum(g, 0.0)
        out_ref[...] = (jnp.dot(r, wf2_ref[...], preferred_element_type=_f32)
                        + bf2_ref[...])


def _head(h3, batch3, pos3, center2d, esm, wf1g, wf1e, bf1, wf2p, bf2p):
    return pl.pallas_call(
        _head_body,
        grid=(N // BN,),
        in_specs=[
            pl.BlockSpec((BN, HID), lambda i: (i, 0)),
            pl.BlockSpec((1, 1, BN), lambda i: (i, 0, 0)),
            pl.BlockSpec((1, 1, BN), lambda i: (i, 0, 0)),
            pl.BlockSpec((B, 128), lambda i: (0, 0)),
            pl.BlockSpec((B, IN), lambda i: (0, 0)),
            pl.BlockSpec((HID, HID), lambda i: (0, 0)),
            pl.BlockSpec((IN, HID), lambda i: (0, 0)),
            pl.BlockSpec((1, HID), lambda i: (0, 0)),
            pl.BlockSpec((HID, 128), lambda i: (0, 0)),
            pl.BlockSpec((1, 128), lambda i: (0, 0)),
        ],
        out_specs=pl.BlockSpec((B, 128), lambda i: (0, 0)),
        out_shape=jax.ShapeDtypeStruct((B, 128), _f32),
        scratch_shapes=[pltpu.VMEM((B, HID), _f32)],
    )(h3, batch3, pos3, center2d, esm, wf1g, wf1e, bf1, wf2p, bf2p)


def kernel(x, edge_index, edge_attr, batch, pos_idx, center_idx, params):
    pad = EP - E
    src = edge_index[0]
    dst = edge_index[1]
    src_pad = jnp.concatenate([src, jnp.zeros((pad,), jnp.int32)])
    srcA = src_pad[:EPH].reshape(32, EPH // 32 // GW, GW)
    srcB = src_pad[EPH:].reshape(32, EPH // 32 // GW, GW)
    dst_pad = jnp.concatenate(
        [dst, N + (jnp.arange(pad, dtype=jnp.int32) & 7)])
    dstA = dst_pad[:EPH].reshape(16, EPH // 16 // SW, SW)
    dstB = dst_pad[EPH:].reshape(16, EPH // 16 // SW, SW)
    zrows = jnp.zeros((ZR, HC), _f32)
    ea_pad = jnp.concatenate([edge_attr, jnp.zeros((pad, ED), _f32)])
    eaA, eaB = ea_pad[:EPH], ea_pad[EPH:]
    batch3 = batch.reshape(N // BN, 1, BN)
    pos3 = pos_idx.reshape(N // BN, 1, BN)
    center2d = jnp.broadcast_to(center_idx[:, None], (B, 128))

    L = params['layers']
    r1 = lambda a: a.reshape(1, -1)
    _gather_x, _gather_h, _scatter_add = _sc_kernels()

    H2 = IN // 2
    xi = lax.bitcast_convert_type(x, jnp.int32) + jnp.int32(0x8000)
    xpk = (lax.shift_right_logical(xi[:, :H2], 16)
           | (xi[:, H2:] & jnp.int32(-65536)))
    We0, be0, W10 = L[0]['We'], L[0]['be'], L[0]['W1']
    ew0 = (We0[:, :H2], We0[:, H2:], r1(be0[:H2]), r1(be0[H2:]),
           W10[:H2], W10[H2:])
    xgA = _gather_x(xpk, srcA)
    m0A = _edge0(xgA, eaA, *ew0)
    xgB = _gather_x(xpk, srcB)
    m0B = _edge0(xgB, eaB, *ew0)
    agg = _scatter_add(m0A, m0B, dstA, dstB, zrows)[:N]
    xw, esm = _xw1_esm(x, L[0]['W1'], batch3, pos3, center2d)
    h = _node0(xw, agg, r1(L[0]['b1']), L[0]['W2'], r1(L[0]['b2']),
               r1(L[0]['gamma']), r1(L[0]['beta']))

    for p in L[1:]:
        hgA = _gather_h(h, srcA)
        mA = _edge(hgA, eaA, p['We'], r1(p['be']))
        hgB = _gather_h(h, srcB)
        mB = _edge(hgB, eaB, p['We'], r1(p['be']))
        agg = _scatter_add(mA, mB, dstA, dstB, zrows)[:N]
        h = _node(h, agg, p['W1'], r1(p['b1']), p['W2'], r1(p['b2']),
                  r1(p['gamma']), r1(p['beta']))

    wf2p = jnp.pad(params['Wf2'], ((0, 0), (0, 126)))
    bf2p = jnp.pad(params['bf2'], (0, 126)).reshape(1, 128)
    out = _head(h, batch3, pos3, center2d, esm,
                params['Wf1'][:HID], params['Wf1'][HID:],
                r1(params['bf1']), wf2p, bf2p)
    return out[:, :2]
